# Initial kernel scaffold; baseline (speedup 1.0000x reference)
import jax
import jax.numpy as jnp
from jax.experimental import pallas as pl


def kernel(real_feats, fake_feats):
    raise NotImplementedError("write your pallas kernel here")


# speedup vs baseline: 21.1783x; 21.1783x over previous
import jax
import jax.numpy as jnp
from jax.experimental import pallas as pl
from jax.experimental.pallas import tpu as pltpu

_KP1 = 4


def _fourth_smallest_sq(d2):
    t = d2
    m = None
    for it in range(_KP1):
        m = jnp.min(t, axis=1, keepdims=True)
        if it < _KP1 - 1:
            t = jnp.where(t <= m, jnp.inf, t)
    return m


def _body(real_ref, fake_ref, rr_ref, rf_ref, met_ref,
          nr_ref, nf_ref, r2r_ref, r2f_ref, prec_ref, rec_ref,
          *, bm, nb, n):
    p = pl.program_id(0)
    i = pl.program_id(1)
    sl = pl.ds(i * bm, bm)

    @pl.when(p == 0)
    def _norms():
        rrow = real_ref[sl, :].astype(jnp.float32)
        nr_ref[0, sl] = jnp.sum(rrow * rrow, axis=1)
        frow = fake_ref[sl, :].astype(jnp.float32)
        nf_ref[0, sl] = jnp.sum(frow * frow, axis=1)

    def _d2_rowblock(rows_bf, cols_ref, colnorm_ref):
        g = jax.lax.dot_general(
            rows_bf, cols_ref[...],
            dimension_numbers=(((1,), (1,)), ((), ())),
            preferred_element_type=jnp.float32)
        rows32 = rows_bf.astype(jnp.float32)
        xn = jnp.sum(rows32 * rows32, axis=1, keepdims=True)
        return xn + colnorm_ref[...] - 2.0 * g

    def _radii_phase(src_ref, norm_ref, radii_out_ref, r2_out_ref):
        d2 = _d2_rowblock(src_ref[sl, :], src_ref, norm_ref)
        v4 = _fourth_smallest_sq(d2)
        r2 = jnp.maximum(v4, 1e-12)
        r2_out_ref[0, sl] = r2[:, 0]
        radii_out_ref[0, sl] = jnp.sqrt(r2)[:, 0]

    @pl.when(p == 1)
    def _real_radii():
        _radii_phase(real_ref, nr_ref, rr_ref, r2r_ref)

    @pl.when(p == 2)
    def _fake_radii():
        _radii_phase(fake_ref, nf_ref, rf_ref, r2f_ref)

    @pl.when(p == 3)
    def _cross():
        d2 = _d2_rowblock(fake_ref[sl, :], real_ref, nr_ref)
        c2 = jnp.maximum(d2, 1e-12)
        within_real = (c2 <= r2r_ref[...]).astype(jnp.float32)
        prec_ref[0, sl] = jnp.max(within_real, axis=1)
        r2f_block = r2f_ref[0, sl].reshape(bm, 1)
        within_fake = (c2 <= r2f_block).astype(jnp.float32)
        rec_part = jnp.max(within_fake, axis=0, keepdims=True)

        @pl.when(i == 0)
        def _():
            rec_ref[...] = rec_part

        @pl.when(i > 0)
        def _():
            rec_ref[...] = jnp.maximum(rec_ref[...], rec_part)

        @pl.when(i == nb - 1)
        def _():
            precision = jnp.sum(prec_ref[...]) / n
            recall = jnp.sum(rec_ref[...]) / n
            lane = jax.lax.broadcasted_iota(jnp.int32, (1, 128), 1)
            met_ref[...] = jnp.where(
                lane == 0, precision, jnp.where(lane == 1, recall, 0.0))


def kernel(real_feats, fake_feats):
    n, d = real_feats.shape
    bm = 256 if n % 256 == 0 else n
    nb = n // bm

    real_bf = real_feats.astype(jnp.bfloat16)
    fake_bf = fake_feats.astype(jnp.bfloat16)

    import functools
    body = functools.partial(_body, bm=bm, nb=nb, n=n)

    full = pl.BlockSpec((n, d), lambda p, i: (0, 0))
    vec = pl.BlockSpec((1, n), lambda p, i: (0, 0))
    met = pl.BlockSpec((1, 128), lambda p, i: (0, 0))

    rr, rf, metrics = pl.pallas_call(
        body,
        grid=(4, nb),
        in_specs=[full, full],
        out_specs=[vec, vec, met],
        out_shape=[
            jax.ShapeDtypeStruct((1, n), jnp.float32),
            jax.ShapeDtypeStruct((1, n), jnp.float32),
            jax.ShapeDtypeStruct((1, 128), jnp.float32),
        ],
        scratch_shapes=[
            pltpu.VMEM((1, n), jnp.float32),
            pltpu.VMEM((1, n), jnp.float32),
            pltpu.VMEM((1, n), jnp.float32),
            pltpu.VMEM((1, n), jnp.float32),
            pltpu.VMEM((1, n), jnp.float32),
            pltpu.VMEM((1, n), jnp.float32),
        ],
        compiler_params=pltpu.CompilerParams(
            dimension_semantics=("arbitrary", "arbitrary")),
    )(real_bf, fake_bf)

    return jnp.concatenate(
        [metrics[0, :2], rr[0, :], rf[0, :]])
